# Initial kernel scaffold; baseline (speedup 1.0000x reference)
import jax
import jax.numpy as jnp
from jax.experimental import pallas as pl


def kernel(x, edge_index, W1, b1, W2, b2):
    raise NotImplementedError("write your pallas kernel here")


# speedup vs baseline: 8.3751x; 8.3751x over previous
import functools

import jax
import jax.numpy as jnp
from jax import lax
from jax.experimental import pallas as pl
from jax.experimental.pallas import tpu as pltpu
from jax.experimental.pallas import tpu_sc as plsc

N_NODES = 10000
N_PAD = 10240
N_EDGES = 320000
CH = 128

NC, NS = 2, 16
NW = NC * NS
C = 128
E_PAD = 327680
EPT = E_PAD // NW
NCH = EPT // C
KCH = 16
NGRP = NCH // KCH
ROWS_PER_TILE = N_PAD // NS
DUMMY = N_PAD - 1


def _fill_vmem(ref, rows, val):
    v = jnp.full((16,), val, jnp.float32)

    def body(i, _):
        for k in range(CH // 16):
            ref[i, pl.ds(k * 16, 16)] = v
        return 0

    lax.fori_loop(0, rows, body, 0)


def _zero_acc_slice(buf, acc, s):
    _fill_vmem(buf, C, 0.0)
    for r in range(ROWS_PER_TILE // C):
        pltpu.sync_copy(buf, acc.at[pl.ds(s * ROWS_PER_TILE + r * C, C)])


def _deg_body(dst_hbm, out_hbm, dst_v, ones_v, acc):
    c = lax.axis_index("c")
    s = lax.axis_index("s")
    g = c * NS + s

    pltpu.sync_copy(dst_hbm.at[g], dst_v)
    _zero_acc_slice(ones_v, acc, s)
    _fill_vmem(ones_v, C, 1.0)
    plsc.subcore_barrier()

    def scat(j, _):
        pltpu.sync_copy(ones_v, acc.at[dst_v.at[j]], add=True)
        return 0

    lax.fori_loop(0, NCH, scat, 0)
    plsc.subcore_barrier()

    pltpu.sync_copy(acc.at[pl.ds(s * ROWS_PER_TILE, ROWS_PER_TILE)],
                    out_hbm.at[g])


def _agg_body(t_hbm, src_hbm, dst_hbm, out_hbm, src_v, dst_v, buf, acc):
    c = lax.axis_index("c")
    s = lax.axis_index("s")
    g = c * NS + s

    _zero_acc_slice(buf, acc, s)
    plsc.subcore_barrier()

    def group(k, _):
        pltpu.sync_copy(src_hbm.at[g, pl.ds(k * KCH, KCH)], src_v)
        pltpu.sync_copy(dst_hbm.at[g, pl.ds(k * KCH, KCH)], dst_v)

        def body(jj, _):
            pltpu.sync_copy(t_hbm.at[src_v.at[jj]], buf)
            pltpu.sync_copy(buf, acc.at[dst_v.at[jj]], add=True)
            return 0

        lax.fori_loop(0, KCH, body, 0)
        return 0

    lax.fori_loop(0, NGRP, group, 0)
    plsc.subcore_barrier()

    pltpu.sync_copy(acc.at[pl.ds(s * ROWS_PER_TILE, ROWS_PER_TILE)],
                    out_hbm.at[g])


@functools.cache
def _sc_kernels():
    mesh = plsc.VectorSubcoreMesh(
        core_axis_name="c", subcore_axis_name="s",
        num_cores=NC, num_subcores=NS)
    out_sds = jax.ShapeDtypeStruct((NW, ROWS_PER_TILE, CH), jnp.float32)
    deg_kernel = pl.kernel(
        _deg_body,
        out_type=out_sds,
        mesh=mesh,
        scratch_types=[
            pltpu.VMEM((NCH, C), jnp.int32),
            pltpu.VMEM((C, CH), jnp.float32),
            pltpu.VMEM_SHARED((N_PAD, CH), jnp.float32),
        ],
    )
    agg_kernel = pl.kernel(
        _agg_body,
        out_type=out_sds,
        mesh=mesh,
        scratch_types=[
            pltpu.VMEM((KCH, C), jnp.int32),
            pltpu.VMEM((KCH, C), jnp.int32),
            pltpu.VMEM((C, CH), jnp.float32),
            pltpu.VMEM_SHARED((N_PAD, CH), jnp.float32),
        ],
    )
    return deg_kernel, agg_kernel


BLK = 1024
GRID = N_PAD // BLK


def _dinv_from(deg_ref):
    d = deg_ref[0, :, 0:1] + deg_ref[1, :, 0:1] + 1.0
    return lax.rsqrt(d)


def _tc_a_body(x_ref, w_ref, deg_ref, o_ref):
    dinv = _dinv_from(deg_ref)
    h = jnp.dot(x_ref[...], w_ref[...], preferred_element_type=jnp.float32)
    o_ref[...] = h * dinv


def _tc_b_body(agg_ref, t_ref, deg_ref, w_ref, b_ref, o_ref):
    dinv = _dinv_from(deg_ref)
    z = (agg_ref[0] + agg_ref[1] + t_ref[...]) * dinv + b_ref[...]
    m = jnp.max(z, axis=1, keepdims=True)
    e = jnp.exp(z - m)
    p = e / jnp.sum(e, axis=1, keepdims=True)
    h = jnp.dot(p, w_ref[...], preferred_element_type=jnp.float32)
    o_ref[...] = h * dinv


def _tc_c_body(agg_ref, t_ref, deg_ref, b_ref, o_ref):
    dinv = _dinv_from(deg_ref)
    o_ref[...] = (agg_ref[0] + agg_ref[1] + t_ref[...]) * dinv + b_ref[...]


_row_spec = pl.BlockSpec((BLK, CH), lambda i: (i, 0))
_agg_spec = pl.BlockSpec((NC, BLK, CH), lambda i: (0, i, 0))
_w_spec = pl.BlockSpec((CH, CH), lambda i: (0, 0))
_b_spec = pl.BlockSpec((1, CH), lambda i: (0, 0))
_out_sds = jax.ShapeDtypeStruct((N_PAD, CH), jnp.float32)

_tc_a = pl.pallas_call(
    _tc_a_body, grid=(GRID,),
    in_specs=[_row_spec, _w_spec, _agg_spec],
    out_specs=_row_spec, out_shape=_out_sds)

_tc_b = pl.pallas_call(
    _tc_b_body, grid=(GRID,),
    in_specs=[_agg_spec, _row_spec, _agg_spec, _w_spec, _b_spec],
    out_specs=_row_spec, out_shape=_out_sds)

_tc_c = pl.pallas_call(
    _tc_c_body, grid=(GRID,),
    in_specs=[_agg_spec, _row_spec, _agg_spec, _b_spec],
    out_specs=_row_spec, out_shape=_out_sds)


@jax.jit
def kernel(x, edge_index, W1, b1, W2, b2):
    _deg_kernel, _agg_kernel = _sc_kernels()
    src = edge_index[0].astype(jnp.int32)
    dst = edge_index[1].astype(jnp.int32)
    pad = jnp.full((E_PAD - N_EDGES,), DUMMY, jnp.int32)
    src3 = jnp.concatenate([src, pad]).reshape(NW, NCH, C)
    dst3 = jnp.concatenate([dst, pad]).reshape(NW, NCH, C)

    x_p = jnp.pad(x, ((0, N_PAD - N_NODES), (0, 0)))
    b1r = b1.reshape(1, CH)
    b2r = b2.reshape(1, CH)

    degp = _deg_kernel(dst3).reshape(NC, N_PAD, CH)
    t1 = _tc_a(x_p, W1, degp)
    agg1 = _agg_kernel(t1, src3, dst3).reshape(NC, N_PAD, CH)
    t2 = _tc_b(agg1, t1, degp, W2, b1r)
    agg2 = _agg_kernel(t2, src3, dst3).reshape(NC, N_PAD, CH)
    out = _tc_c(agg2, t2, degp, b2r)
    return out[:N_NODES]


# speedup vs baseline: 8.4061x; 1.0037x over previous
import functools

import jax
import jax.numpy as jnp
from jax import lax
from jax.experimental import pallas as pl
from jax.experimental.pallas import tpu as pltpu
from jax.experimental.pallas import tpu_sc as plsc

N_NODES = 10000
N_PAD = 10240
N_EDGES = 320000
CH = 128

NC, NS = 2, 16
NW = NC * NS
C = 128
E_PAD = 327680
EPT = E_PAD // NW
NCH = EPT // C
NB = 2
KCH = 16
NGRP = NCH // KCH
ROWS_PER_TILE = N_PAD // NS
DUMMY = N_PAD - 1


def _fill_vmem(ref, rows, val):
    v = jnp.full((16,), val, jnp.float32)

    def body(i, _):
        for k in range(CH // 16):
            ref[i, pl.ds(k * 16, 16)] = v
        return 0

    lax.fori_loop(0, rows, body, 0)


def _zero_acc_slice(buf, acc, s):
    _fill_vmem(buf, C, 0.0)
    for r in range(ROWS_PER_TILE // C):
        pltpu.sync_copy(buf, acc.at[pl.ds(s * ROWS_PER_TILE + r * C, C)])


def _deg_body(dst_hbm, out_hbm, dst_v, ones_v, acc):
    c = lax.axis_index("c")
    s = lax.axis_index("s")
    g = c * NS + s

    pltpu.sync_copy(dst_hbm.at[g], dst_v)
    _zero_acc_slice(ones_v, acc, s)
    _fill_vmem(ones_v, C, 1.0)
    plsc.subcore_barrier()

    def scat(j, _):
        pltpu.sync_copy(ones_v, acc.at[dst_v.at[j]], add=True)
        return 0

    lax.fori_loop(0, NCH, scat, 0)
    plsc.subcore_barrier()

    pltpu.sync_copy(acc.at[pl.ds(s * ROWS_PER_TILE, ROWS_PER_TILE)],
                    out_hbm.at[g])


def _agg_body(t_hbm, src_hbm, dst_hbm, out_hbm, src_v, dst_v,
              b0, b1, sem, acc):
    c = lax.axis_index("c")
    s = lax.axis_index("s")
    g = c * NS + s
    bufs = [b0, b1]

    _zero_acc_slice(b0, acc, s)
    plsc.subcore_barrier()

    def group(k, _):
        pltpu.sync_copy(src_hbm.at[g, pl.ds(k * KCH, KCH)], src_v)
        pltpu.sync_copy(dst_hbm.at[g, pl.ds(k * KCH, KCH)], dst_v)

        for b in range(NB):
            pltpu.async_copy(t_hbm.at[src_v.at[b]], bufs[b], sem)

        def turn(t, _):
            for b in range(NB):
                j = t * NB + b
                pltpu.make_async_copy(t_hbm.at[src_v.at[0]],
                                      bufs[b], sem).wait()
                pltpu.sync_copy(bufs[b], acc.at[dst_v.at[j]], add=True)
                jn = jnp.minimum(j + NB, KCH - 1)
                pltpu.async_copy(t_hbm.at[src_v.at[jn]], bufs[b], sem)
            return 0

        lax.fori_loop(0, KCH // NB, turn, 0)

        for b in range(NB):
            pltpu.make_async_copy(t_hbm.at[src_v.at[0]], bufs[b], sem).wait()
        return 0

    lax.fori_loop(0, NGRP, group, 0)
    plsc.subcore_barrier()

    pltpu.sync_copy(acc.at[pl.ds(s * ROWS_PER_TILE, ROWS_PER_TILE)],
                    out_hbm.at[g])


@functools.cache
def _sc_kernels():
    mesh = plsc.VectorSubcoreMesh(
        core_axis_name="c", subcore_axis_name="s",
        num_cores=NC, num_subcores=NS)
    out_sds = jax.ShapeDtypeStruct((NW, ROWS_PER_TILE, CH), jnp.float32)
    deg_kernel = pl.kernel(
        _deg_body,
        out_type=out_sds,
        mesh=mesh,
        scratch_types=[
            pltpu.VMEM((NCH, C), jnp.int32),
            pltpu.VMEM((C, CH), jnp.float32),
            pltpu.VMEM_SHARED((N_PAD, CH), jnp.float32),
        ],
    )
    agg_kernel = pl.kernel(
        _agg_body,
        out_type=out_sds,
        mesh=mesh,
        scratch_types=[
            pltpu.VMEM((KCH, C), jnp.int32),
            pltpu.VMEM((KCH, C), jnp.int32),
            pltpu.VMEM((C, CH), jnp.float32),
            pltpu.VMEM((C, CH), jnp.float32),
            pltpu.SemaphoreType.DMA,
            pltpu.VMEM_SHARED((N_PAD, CH), jnp.float32),
        ],
    )
    return deg_kernel, agg_kernel


BLK = 1024
GRID = N_PAD // BLK


def _dinv_from(deg_ref):
    d = deg_ref[0, :, 0:1] + deg_ref[1, :, 0:1] + 1.0
    return lax.rsqrt(d)


def _tc_a_body(x_ref, w_ref, deg_ref, o_ref):
    dinv = _dinv_from(deg_ref)
    h = jnp.dot(x_ref[...], w_ref[...], preferred_element_type=jnp.float32)
    o_ref[...] = h * dinv


def _tc_b_body(agg_ref, t_ref, deg_ref, w_ref, b_ref, o_ref):
    dinv = _dinv_from(deg_ref)
    z = (agg_ref[0] + agg_ref[1] + t_ref[...]) * dinv + b_ref[...]
    m = jnp.max(z, axis=1, keepdims=True)
    e = jnp.exp(z - m)
    p = e / jnp.sum(e, axis=1, keepdims=True)
    h = jnp.dot(p, w_ref[...], preferred_element_type=jnp.float32)
    o_ref[...] = h * dinv


def _tc_c_body(agg_ref, t_ref, deg_ref, b_ref, o_ref):
    dinv = _dinv_from(deg_ref)
    o_ref[...] = (agg_ref[0] + agg_ref[1] + t_ref[...]) * dinv + b_ref[...]


_row_spec = pl.BlockSpec((BLK, CH), lambda i: (i, 0))
_agg_spec = pl.BlockSpec((NC, BLK, CH), lambda i: (0, i, 0))
_w_spec = pl.BlockSpec((CH, CH), lambda i: (0, 0))
_b_spec = pl.BlockSpec((1, CH), lambda i: (0, 0))
_out_sds = jax.ShapeDtypeStruct((N_PAD, CH), jnp.float32)

_tc_a = pl.pallas_call(
    _tc_a_body, grid=(GRID,),
    in_specs=[_row_spec, _w_spec, _agg_spec],
    out_specs=_row_spec, out_shape=_out_sds)

_tc_b = pl.pallas_call(
    _tc_b_body, grid=(GRID,),
    in_specs=[_agg_spec, _row_spec, _agg_spec, _w_spec, _b_spec],
    out_specs=_row_spec, out_shape=_out_sds)

_tc_c = pl.pallas_call(
    _tc_c_body, grid=(GRID,),
    in_specs=[_agg_spec, _row_spec, _agg_spec, _b_spec],
    out_specs=_row_spec, out_shape=_out_sds)


@jax.jit
def kernel(x, edge_index, W1, b1, W2, b2):
    _deg_kernel, _agg_kernel = _sc_kernels()
    src = edge_index[0].astype(jnp.int32)
    dst = edge_index[1].astype(jnp.int32)
    pad = jnp.full((E_PAD - N_EDGES,), DUMMY, jnp.int32)
    src3 = jnp.concatenate([src, pad]).reshape(NW, NCH, C)
    dst3 = jnp.concatenate([dst, pad]).reshape(NW, NCH, C)

    x_p = jnp.pad(x, ((0, N_PAD - N_NODES), (0, 0)))
    b1r = b1.reshape(1, CH)
    b2r = b2.reshape(1, CH)

    degp = _deg_kernel(dst3).reshape(NC, N_PAD, CH)
    t1 = _tc_a(x_p, W1, degp)
    agg1 = _agg_kernel(t1, src3, dst3).reshape(NC, N_PAD, CH)
    t2 = _tc_b(agg1, t1, degp, W2, b1r)
    agg2 = _agg_kernel(t2, src3, dst3).reshape(NC, N_PAD, CH)
    out = _tc_c(agg2, t2, degp, b2r)
    return out[:N_NODES]


# speedup vs baseline: 8.5411x; 1.0160x over previous
import functools

import jax
import jax.numpy as jnp
from jax import lax
from jax.experimental import pallas as pl
from jax.experimental.pallas import tpu as pltpu
from jax.experimental.pallas import tpu_sc as plsc

N_NODES = 10000
N_PAD = 10240
N_EDGES = 320000
CH = 128

NC, NS = 2, 16
NW = NC * NS
C = 128
E_PAD = 327680
EPT = E_PAD // NW
NCH = EPT // C
NB = 2
KCH = 16
NGRP = NCH // KCH
ROWS_PER_TILE = N_PAD // NS
DUMMY = N_PAD - 1


def _fill_vmem(ref, rows, val):
    v = jnp.full((16,), val, jnp.float32)

    def body(i, _):
        for k in range(CH // 16):
            ref[i, pl.ds(k * 16, 16)] = v
        return 0

    lax.fori_loop(0, rows, body, 0)


def _zero_acc_slice(buf, acc, s):
    _fill_vmem(buf, C, 0.0)
    for r in range(ROWS_PER_TILE // C):
        pltpu.sync_copy(buf, acc.at[pl.ds(s * ROWS_PER_TILE + r * C, C)])


def _deg_body(dst_hbm, out_hbm, dst_v, ones_v, acc):
    c = lax.axis_index("c")
    s = lax.axis_index("s")
    g = c * NS + s

    pltpu.sync_copy(dst_hbm.at[g], dst_v)
    _zero_acc_slice(ones_v, acc, s)
    _fill_vmem(ones_v, C, 1.0)
    plsc.subcore_barrier()

    def scat(j, _):
        pltpu.sync_copy(ones_v, acc.at[dst_v.at[j]], add=True)
        return 0

    lax.fori_loop(0, NCH, scat, 0)
    plsc.subcore_barrier()

    pltpu.sync_copy(acc.at[pl.ds(s * ROWS_PER_TILE, ROWS_PER_TILE)],
                    out_hbm.at[g])


def _agg_body(t_hbm, src_hbm, dst_hbm, out_hbm, src_v, dst_v,
              b0, b1, sem_g, sem_s, acc):
    c = lax.axis_index("c")
    s = lax.axis_index("s")
    g = c * NS + s
    bufs = [b0, b1]

    _zero_acc_slice(b0, acc, s)
    plsc.subcore_barrier()

    def group(k, _):
        pltpu.sync_copy(src_hbm.at[g, pl.ds(k * KCH, KCH)], src_v)
        pltpu.sync_copy(dst_hbm.at[g, pl.ds(k * KCH, KCH)], dst_v)

        pltpu.async_copy(t_hbm.at[src_v.at[0]], bufs[0], sem_g)
        for j in range(KCH):
            b = j % 2
            ob = 1 - b
            pltpu.make_async_copy(t_hbm.at[src_v.at[0]],
                                  bufs[b], sem_g).wait()
            pltpu.async_copy(bufs[b], acc.at[dst_v.at[j]], sem_s, add=True)
            if j >= 1:
                pltpu.make_async_copy(bufs[ob], acc.at[dst_v.at[0]],
                                      sem_s).wait()
            jn = j + 1 if j + 1 < KCH else KCH - 1
            pltpu.async_copy(t_hbm.at[src_v.at[jn]], bufs[ob], sem_g)

        pltpu.make_async_copy(t_hbm.at[src_v.at[0]], bufs[0], sem_g).wait()
        pltpu.make_async_copy(bufs[1], acc.at[dst_v.at[0]],
                              sem_s).wait()
        return 0

    lax.fori_loop(0, NGRP, group, 0)
    plsc.subcore_barrier()

    pltpu.sync_copy(acc.at[pl.ds(s * ROWS_PER_TILE, ROWS_PER_TILE)],
                    out_hbm.at[g])


@functools.cache
def _sc_kernels():
    mesh = plsc.VectorSubcoreMesh(
        core_axis_name="c", subcore_axis_name="s",
        num_cores=NC, num_subcores=NS)
    out_sds = jax.ShapeDtypeStruct((NW, ROWS_PER_TILE, CH), jnp.float32)
    deg_kernel = pl.kernel(
        _deg_body,
        out_type=out_sds,
        mesh=mesh,
        scratch_types=[
            pltpu.VMEM((NCH, C), jnp.int32),
            pltpu.VMEM((C, CH), jnp.float32),
            pltpu.VMEM_SHARED((N_PAD, CH), jnp.float32),
        ],
    )
    agg_kernel = pl.kernel(
        _agg_body,
        out_type=out_sds,
        mesh=mesh,
        scratch_types=[
            pltpu.VMEM((KCH, C), jnp.int32),
            pltpu.VMEM((KCH, C), jnp.int32),
            pltpu.VMEM((C, CH), jnp.float32),
            pltpu.VMEM((C, CH), jnp.float32),
            pltpu.SemaphoreType.DMA,
            pltpu.SemaphoreType.DMA,
            pltpu.VMEM_SHARED((N_PAD, CH), jnp.float32),
        ],
    )
    return deg_kernel, agg_kernel


BLK = 1024
GRID = N_PAD // BLK


def _dinv_from(deg_ref):
    d = deg_ref[0, :, 0:1] + deg_ref[1, :, 0:1] + 1.0
    return lax.rsqrt(d)


def _tc_a_body(x_ref, w_ref, deg_ref, o_ref):
    dinv = _dinv_from(deg_ref)
    h = jnp.dot(x_ref[...], w_ref[...], preferred_element_type=jnp.float32)
    o_ref[...] = h * dinv


def _tc_b_body(agg_ref, t_ref, deg_ref, w_ref, b_ref, o_ref):
    dinv = _dinv_from(deg_ref)
    z = (agg_ref[0] + agg_ref[1] + t_ref[...]) * dinv + b_ref[...]
    m = jnp.max(z, axis=1, keepdims=True)
    e = jnp.exp(z - m)
    p = e / jnp.sum(e, axis=1, keepdims=True)
    h = jnp.dot(p, w_ref[...], preferred_element_type=jnp.float32)
    o_ref[...] = h * dinv


def _tc_c_body(agg_ref, t_ref, deg_ref, b_ref, o_ref):
    dinv = _dinv_from(deg_ref)
    o_ref[...] = (agg_ref[0] + agg_ref[1] + t_ref[...]) * dinv + b_ref[...]


_row_spec = pl.BlockSpec((BLK, CH), lambda i: (i, 0))
_agg_spec = pl.BlockSpec((NC, BLK, CH), lambda i: (0, i, 0))
_w_spec = pl.BlockSpec((CH, CH), lambda i: (0, 0))
_b_spec = pl.BlockSpec((1, CH), lambda i: (0, 0))
_out_sds = jax.ShapeDtypeStruct((N_PAD, CH), jnp.float32)

_tc_a = pl.pallas_call(
    _tc_a_body, grid=(GRID,),
    in_specs=[_row_spec, _w_spec, _agg_spec],
    out_specs=_row_spec, out_shape=_out_sds)

_tc_b = pl.pallas_call(
    _tc_b_body, grid=(GRID,),
    in_specs=[_agg_spec, _row_spec, _agg_spec, _w_spec, _b_spec],
    out_specs=_row_spec, out_shape=_out_sds)

_tc_c = pl.pallas_call(
    _tc_c_body, grid=(GRID,),
    in_specs=[_agg_spec, _row_spec, _agg_spec, _b_spec],
    out_specs=_row_spec, out_shape=_out_sds)


@jax.jit
def kernel(x, edge_index, W1, b1, W2, b2):
    _deg_kernel, _agg_kernel = _sc_kernels()
    src = edge_index[0].astype(jnp.int32)
    dst = edge_index[1].astype(jnp.int32)
    pad = jnp.full((E_PAD - N_EDGES,), DUMMY, jnp.int32)
    src3 = jnp.concatenate([src, pad]).reshape(NW, NCH, C)
    dst3 = jnp.concatenate([dst, pad]).reshape(NW, NCH, C)

    x_p = jnp.pad(x, ((0, N_PAD - N_NODES), (0, 0)))
    b1r = b1.reshape(1, CH)
    b2r = b2.reshape(1, CH)

    degp = _deg_kernel(dst3).reshape(NC, N_PAD, CH)
    t1 = _tc_a(x_p, W1, degp)
    agg1 = _agg_kernel(t1, src3, dst3).reshape(NC, N_PAD, CH)
    t2 = _tc_b(agg1, t1, degp, W2, b1r)
    agg2 = _agg_kernel(t2, src3, dst3).reshape(NC, N_PAD, CH)
    out = _tc_c(agg2, t2, degp, b2r)
    return out[:N_NODES]


# speedup vs baseline: 8.8513x; 1.0363x over previous
import functools

import jax
import jax.numpy as jnp
from jax import lax
from jax.experimental import pallas as pl
from jax.experimental.pallas import tpu as pltpu
from jax.experimental.pallas import tpu_sc as plsc

N_NODES = 10000
N_PAD = 10240
N_EDGES = 320000
CH = 128

NC, NS = 2, 16
NW = NC * NS
C = 128
E_PAD = 327680
EPT = E_PAD // NW
NCH = EPT // C
NB = 2
KCH = 16
NGRP = NCH // KCH
ROWS_PER_TILE = N_PAD // NS
DUMMY = N_PAD - 1


DCH = 16


def _fill_vmem(ref, rows, width, val):
    v = jnp.full((16,), val, jnp.float32)

    def body(i, _):
        for k in range(width // 16):
            ref[i, pl.ds(k * 16, 16)] = v
        return 0

    lax.fori_loop(0, rows, body, 0)


def _zero_acc_slice(buf, acc, s, width):
    _fill_vmem(buf, C, width, 0.0)
    for r in range(ROWS_PER_TILE // C):
        pltpu.sync_copy(buf, acc.at[pl.ds(s * ROWS_PER_TILE + r * C, C)])


def _deg_body(dst_hbm, out_hbm, dst_v, ones_v, acc):
    c = lax.axis_index("c")
    s = lax.axis_index("s")
    g = c * NS + s

    pltpu.sync_copy(dst_hbm.at[g], dst_v)
    _zero_acc_slice(ones_v, acc, s, DCH)
    _fill_vmem(ones_v, C, DCH, 1.0)
    plsc.subcore_barrier()

    def scat(j, _):
        pltpu.sync_copy(ones_v, acc.at[dst_v.at[j]], add=True)
        return 0

    lax.fori_loop(0, NCH, scat, 0)
    plsc.subcore_barrier()

    pltpu.sync_copy(acc.at[pl.ds(s * ROWS_PER_TILE, ROWS_PER_TILE)],
                    out_hbm.at[g])


def _agg_body(t_hbm, src_hbm, dst_hbm, out_hbm, src_v, dst_v,
              b0, b1, sem_g, sem_s, acc):
    c = lax.axis_index("c")
    s = lax.axis_index("s")
    g = c * NS + s
    bufs = [b0, b1]

    _zero_acc_slice(b0, acc, s, CH)
    plsc.subcore_barrier()

    def group(k, _):
        pltpu.sync_copy(src_hbm.at[g, pl.ds(k * KCH, KCH)], src_v)
        pltpu.sync_copy(dst_hbm.at[g, pl.ds(k * KCH, KCH)], dst_v)

        pltpu.async_copy(t_hbm.at[src_v.at[0]], bufs[0], sem_g)
        for j in range(KCH):
            b = j % 2
            ob = 1 - b
            pltpu.make_async_copy(t_hbm.at[src_v.at[0]],
                                  bufs[b], sem_g).wait()
            pltpu.async_copy(bufs[b], acc.at[dst_v.at[j]], sem_s, add=True)
            if j >= 1:
                pltpu.make_async_copy(bufs[ob], acc.at[dst_v.at[0]],
                                      sem_s).wait()
            jn = j + 1 if j + 1 < KCH else KCH - 1
            pltpu.async_copy(t_hbm.at[src_v.at[jn]], bufs[ob], sem_g)

        pltpu.make_async_copy(t_hbm.at[src_v.at[0]], bufs[0], sem_g).wait()
        pltpu.make_async_copy(bufs[1], acc.at[dst_v.at[0]],
                              sem_s).wait()
        return 0

    lax.fori_loop(0, NGRP, group, 0)
    plsc.subcore_barrier()

    pltpu.sync_copy(acc.at[pl.ds(s * ROWS_PER_TILE, ROWS_PER_TILE)],
                    out_hbm.at[g])


@functools.cache
def _sc_kernels():
    mesh = plsc.VectorSubcoreMesh(
        core_axis_name="c", subcore_axis_name="s",
        num_cores=NC, num_subcores=NS)
    out_sds = jax.ShapeDtypeStruct((NW, ROWS_PER_TILE, CH), jnp.float32)
    deg_kernel = pl.kernel(
        _deg_body,
        out_type=jax.ShapeDtypeStruct((NW, ROWS_PER_TILE, DCH), jnp.float32),
        mesh=mesh,
        scratch_types=[
            pltpu.VMEM((NCH, C), jnp.int32),
            pltpu.VMEM((C, DCH), jnp.float32),
            pltpu.VMEM_SHARED((N_PAD, DCH), jnp.float32),
        ],
    )
    agg_kernel = pl.kernel(
        _agg_body,
        out_type=out_sds,
        mesh=mesh,
        scratch_types=[
            pltpu.VMEM((KCH, C), jnp.int32),
            pltpu.VMEM((KCH, C), jnp.int32),
            pltpu.VMEM((C, CH), jnp.float32),
            pltpu.VMEM((C, CH), jnp.float32),
            pltpu.SemaphoreType.DMA,
            pltpu.SemaphoreType.DMA,
            pltpu.VMEM_SHARED((N_PAD, CH), jnp.float32),
        ],
    )
    return deg_kernel, agg_kernel


BLK = 1024
GRID = N_PAD // BLK


def _dinv_from(deg_ref):
    d = deg_ref[0, :, 0:1] + deg_ref[1, :, 0:1] + 1.0
    return lax.rsqrt(d)


def _tc_a_body(x_ref, w_ref, deg_ref, o_ref):
    dinv = _dinv_from(deg_ref)
    h = jnp.dot(x_ref[...], w_ref[...], preferred_element_type=jnp.float32)
    o_ref[...] = h * dinv


def _tc_b_body(agg_ref, t_ref, deg_ref, w_ref, b_ref, o_ref):
    dinv = _dinv_from(deg_ref)
    z = (agg_ref[0] + agg_ref[1] + t_ref[...]) * dinv + b_ref[...]
    m = jnp.max(z, axis=1, keepdims=True)
    e = jnp.exp(z - m)
    p = e / jnp.sum(e, axis=1, keepdims=True)
    h = jnp.dot(p, w_ref[...], preferred_element_type=jnp.float32)
    o_ref[...] = h * dinv


def _tc_c_body(agg_ref, t_ref, deg_ref, b_ref, o_ref):
    dinv = _dinv_from(deg_ref)
    o_ref[...] = (agg_ref[0] + agg_ref[1] + t_ref[...]) * dinv + b_ref[...]


_row_spec = pl.BlockSpec((BLK, CH), lambda i: (i, 0))
_agg_spec = pl.BlockSpec((NC, BLK, CH), lambda i: (0, i, 0))
_deg_spec = pl.BlockSpec((NC, BLK, DCH), lambda i: (0, i, 0))
_w_spec = pl.BlockSpec((CH, CH), lambda i: (0, 0))
_b_spec = pl.BlockSpec((1, CH), lambda i: (0, 0))
_out_sds = jax.ShapeDtypeStruct((N_PAD, CH), jnp.float32)

_tc_a = pl.pallas_call(
    _tc_a_body, grid=(GRID,),
    in_specs=[_row_spec, _w_spec, _deg_spec],
    out_specs=_row_spec, out_shape=_out_sds)

_tc_b = pl.pallas_call(
    _tc_b_body, grid=(GRID,),
    in_specs=[_agg_spec, _row_spec, _deg_spec, _w_spec, _b_spec],
    out_specs=_row_spec, out_shape=_out_sds)

_tc_c = pl.pallas_call(
    _tc_c_body, grid=(GRID,),
    in_specs=[_agg_spec, _row_spec, _deg_spec, _b_spec],
    out_specs=_row_spec, out_shape=_out_sds)


@jax.jit
def kernel(x, edge_index, W1, b1, W2, b2):
    _deg_kernel, _agg_kernel = _sc_kernels()
    src = edge_index[0].astype(jnp.int32)
    dst = edge_index[1].astype(jnp.int32)
    pad = jnp.full((E_PAD - N_EDGES,), DUMMY, jnp.int32)
    src3 = jnp.concatenate([src, pad]).reshape(NW, NCH, C)
    dst3 = jnp.concatenate([dst, pad]).reshape(NW, NCH, C)

    x_p = jnp.pad(x, ((0, N_PAD - N_NODES), (0, 0)))
    b1r = b1.reshape(1, CH)
    b2r = b2.reshape(1, CH)

    degp = _deg_kernel(dst3).reshape(NC, N_PAD, DCH)
    t1 = _tc_a(x_p, W1, degp)
    agg1 = _agg_kernel(t1, src3, dst3).reshape(NC, N_PAD, CH)
    t2 = _tc_b(agg1, t1, degp, W2, b1r)
    agg2 = _agg_kernel(t2, src3, dst3).reshape(NC, N_PAD, CH)
    out = _tc_c(agg2, t2, degp, b2r)
    return out[:N_NODES]


# speedup vs baseline: 9.1519x; 1.0340x over previous
import functools

import jax
import jax.numpy as jnp
from jax import lax
from jax.experimental import pallas as pl
from jax.experimental.pallas import tpu as pltpu
from jax.experimental.pallas import tpu_sc as plsc

N_NODES = 10000
N_PAD = 10240
N_EDGES = 320000
CH = 128

NC, NS = 2, 16
NW = NC * NS
C = 128
E_PAD = 327680
EPT = E_PAD // NW
NCH = EPT // C
NB = 2
KCH = 40
NGRP = NCH // KCH
ROWS_PER_TILE = N_PAD // NS
DUMMY = N_PAD - 1


DCH = 16


def _fill_vmem(ref, rows, width, val):
    v = jnp.full((16,), val, jnp.float32)

    def body(i, _):
        for k in range(width // 16):
            ref[i, pl.ds(k * 16, 16)] = v
        return 0

    lax.fori_loop(0, rows, body, 0)


def _zero_acc_slice(buf, acc, s, width):
    _fill_vmem(buf, C, width, 0.0)
    for r in range(ROWS_PER_TILE // C):
        pltpu.sync_copy(buf, acc.at[pl.ds(s * ROWS_PER_TILE + r * C, C)])


def _deg_body(dst_hbm, out_hbm, dst_v, ones_v, acc):
    c = lax.axis_index("c")
    s = lax.axis_index("s")
    g = c * NS + s

    pltpu.sync_copy(dst_hbm.at[g], dst_v)
    _zero_acc_slice(ones_v, acc, s, DCH)
    _fill_vmem(ones_v, C, DCH, 1.0)
    plsc.subcore_barrier()

    def scat(j, _):
        pltpu.sync_copy(ones_v, acc.at[dst_v.at[j]], add=True)
        return 0

    lax.fori_loop(0, NCH, scat, 0)
    plsc.subcore_barrier()

    pltpu.sync_copy(acc.at[pl.ds(s * ROWS_PER_TILE, ROWS_PER_TILE)],
                    out_hbm.at[g])


def _agg_body(t_hbm, src_hbm, dst_hbm, out_hbm, src_v, dst_v,
              b0, b1, sem_g, sem_s, acc):
    c = lax.axis_index("c")
    s = lax.axis_index("s")
    g = c * NS + s
    bufs = [b0, b1]

    _zero_acc_slice(b0, acc, s, CH)
    plsc.subcore_barrier()

    def group(k, _):
        pltpu.sync_copy(src_hbm.at[g, pl.ds(k * KCH, KCH)], src_v)
        pltpu.sync_copy(dst_hbm.at[g, pl.ds(k * KCH, KCH)], dst_v)

        pltpu.async_copy(t_hbm.at[src_v.at[0]], bufs[0], sem_g)
        for j in range(KCH):
            b = j % 2
            ob = 1 - b
            pltpu.make_async_copy(t_hbm.at[src_v.at[0]],
                                  bufs[b], sem_g).wait()
            pltpu.async_copy(bufs[b], acc.at[dst_v.at[j]], sem_s, add=True)
            if j >= 1:
                pltpu.make_async_copy(bufs[ob], acc.at[dst_v.at[0]],
                                      sem_s).wait()
            jn = j + 1 if j + 1 < KCH else KCH - 1
            pltpu.async_copy(t_hbm.at[src_v.at[jn]], bufs[ob], sem_g)

        pltpu.make_async_copy(t_hbm.at[src_v.at[0]], bufs[0], sem_g).wait()
        pltpu.make_async_copy(bufs[1], acc.at[dst_v.at[0]],
                              sem_s).wait()
        return 0

    lax.fori_loop(0, NGRP, group, 0)
    plsc.subcore_barrier()

    pltpu.sync_copy(acc.at[pl.ds(s * ROWS_PER_TILE, ROWS_PER_TILE)],
                    out_hbm.at[g])


@functools.cache
def _sc_kernels():
    mesh = plsc.VectorSubcoreMesh(
        core_axis_name="c", subcore_axis_name="s",
        num_cores=NC, num_subcores=NS)
    out_sds = jax.ShapeDtypeStruct((NW, ROWS_PER_TILE, CH), jnp.float32)
    deg_kernel = pl.kernel(
        _deg_body,
        out_type=jax.ShapeDtypeStruct((NW, ROWS_PER_TILE, DCH), jnp.float32),
        mesh=mesh,
        scratch_types=[
            pltpu.VMEM((NCH, C), jnp.int32),
            pltpu.VMEM((C, DCH), jnp.float32),
            pltpu.VMEM_SHARED((N_PAD, DCH), jnp.float32),
        ],
    )
    agg_kernel = pl.kernel(
        _agg_body,
        out_type=out_sds,
        mesh=mesh,
        scratch_types=[
            pltpu.VMEM((KCH, C), jnp.int32),
            pltpu.VMEM((KCH, C), jnp.int32),
            pltpu.VMEM((C, CH), jnp.float32),
            pltpu.VMEM((C, CH), jnp.float32),
            pltpu.SemaphoreType.DMA,
            pltpu.SemaphoreType.DMA,
            pltpu.VMEM_SHARED((N_PAD, CH), jnp.float32),
        ],
    )
    return deg_kernel, agg_kernel


BLK = 1024
GRID = N_PAD // BLK


def _dinv_from(deg_ref):
    d = deg_ref[0, :, 0:1] + deg_ref[1, :, 0:1] + 1.0
    return lax.rsqrt(d)


def _tc_a_body(x_ref, w_ref, deg_ref, o_ref):
    dinv = _dinv_from(deg_ref)
    h = jnp.dot(x_ref[...], w_ref[...], preferred_element_type=jnp.float32)
    o_ref[...] = h * dinv


def _tc_b_body(agg_ref, t_ref, deg_ref, w_ref, b_ref, o_ref):
    dinv = _dinv_from(deg_ref)
    z = (agg_ref[0] + agg_ref[1] + t_ref[...]) * dinv + b_ref[...]
    m = jnp.max(z, axis=1, keepdims=True)
    e = jnp.exp(z - m)
    p = e / jnp.sum(e, axis=1, keepdims=True)
    h = jnp.dot(p, w_ref[...], preferred_element_type=jnp.float32)
    o_ref[...] = h * dinv


def _tc_c_body(agg_ref, t_ref, deg_ref, b_ref, o_ref):
    dinv = _dinv_from(deg_ref)
    o_ref[...] = (agg_ref[0] + agg_ref[1] + t_ref[...]) * dinv + b_ref[...]


_row_spec = pl.BlockSpec((BLK, CH), lambda i: (i, 0))
_agg_spec = pl.BlockSpec((NC, BLK, CH), lambda i: (0, i, 0))
_deg_spec = pl.BlockSpec((NC, BLK, DCH), lambda i: (0, i, 0))
_w_spec = pl.BlockSpec((CH, CH), lambda i: (0, 0))
_b_spec = pl.BlockSpec((1, CH), lambda i: (0, 0))
_out_sds = jax.ShapeDtypeStruct((N_PAD, CH), jnp.float32)

_tc_a = pl.pallas_call(
    _tc_a_body, grid=(GRID,),
    in_specs=[_row_spec, _w_spec, _deg_spec],
    out_specs=_row_spec, out_shape=_out_sds)

_tc_b = pl.pallas_call(
    _tc_b_body, grid=(GRID,),
    in_specs=[_agg_spec, _row_spec, _deg_spec, _w_spec, _b_spec],
    out_specs=_row_spec, out_shape=_out_sds)

_tc_c = pl.pallas_call(
    _tc_c_body, grid=(GRID,),
    in_specs=[_agg_spec, _row_spec, _deg_spec, _b_spec],
    out_specs=_row_spec, out_shape=_out_sds)


@jax.jit
def kernel(x, edge_index, W1, b1, W2, b2):
    _deg_kernel, _agg_kernel = _sc_kernels()
    src = edge_index[0].astype(jnp.int32)
    dst = edge_index[1].astype(jnp.int32)
    pad = jnp.full((E_PAD - N_EDGES,), DUMMY, jnp.int32)
    src3 = jnp.concatenate([src, pad]).reshape(NW, NCH, C)
    dst3 = jnp.concatenate([dst, pad]).reshape(NW, NCH, C)

    x_p = jnp.pad(x, ((0, N_PAD - N_NODES), (0, 0)))
    b1r = b1.reshape(1, CH)
    b2r = b2.reshape(1, CH)

    degp = _deg_kernel(dst3).reshape(NC, N_PAD, DCH)
    t1 = _tc_a(x_p, W1, degp)
    agg1 = _agg_kernel(t1, src3, dst3).reshape(NC, N_PAD, CH)
    t2 = _tc_b(agg1, t1, degp, W2, b1r)
    agg2 = _agg_kernel(t2, src3, dst3).reshape(NC, N_PAD, CH)
    out = _tc_c(agg2, t2, degp, b2r)
    return out[:N_NODES]
